# baseline (speedup 1.0000x reference)
import functools

import jax
import jax.numpy as jnp
from jax import lax
from jax.experimental import pallas as pl
from jax.experimental.pallas import tpu as pltpu
from jax.experimental.pallas import tpu_sc as plsc


def kernel(embedded_features, targets, hsa):
    B, D = embedded_features.shape
    C = hsa.shape[0]
    info = plsc.get_sparse_core_info()
    NC, NS, L = info.num_cores, info.num_subcores, info.num_lanes
    NW = NC * NS
    b_per_w = B // NW

    mesh = plsc.VectorSubcoreMesh(core_axis_name="c", subcore_axis_name="s")

    @functools.partial(
        pl.kernel,
        mesh=mesh,
        compiler_params=pltpu.CompilerParams(needs_layout_passes=False),
        out_type=jax.ShapeDtypeStruct((NW, L), jnp.float32),
        scratch_types=[
            pltpu.VMEM((b_per_w,), jnp.int32),
            pltpu.VMEM((D, b_per_w), jnp.float32),
            pltpu.VMEM((D, C), jnp.float32),
            pltpu.VMEM((L,), jnp.float32),
            pltpu.SemaphoreType.DMA,
            pltpu.SemaphoreType.DMA,
            pltpu.SemaphoreType.DMA,
            pltpu.SemaphoreType.DMA,
        ],
    )
    def run(
        feat_hbm, tgt_hbm, hsa_hbm, out_hbm,
        idx_v, feat_v, tbl_v, acc_v, sem_t, sem_i, sem_f0, sem_f1,
    ):
        wid = lax.axis_index("s") * NC + lax.axis_index("c")
        base = wid * b_per_w
        half = b_per_w // 2
        ct = pltpu.async_copy(hsa_hbm, tbl_v, sem_t)
        ci = pltpu.async_copy(tgt_hbm.at[pl.ds(base, b_per_w)], idx_v, sem_i)
        cf0 = pltpu.async_copy(
            feat_hbm.at[:, pl.ds(base, half)], feat_v.at[:, pl.ds(0, half)], sem_f0
        )
        cf1 = pltpu.async_copy(
            feat_hbm.at[:, pl.ds(base + half, half)],
            feat_v.at[:, pl.ds(half, half)],
            sem_f1,
        )

        zero = jnp.zeros((L,), jnp.float32)
        n_acc = 8

        def make_loop(lo, hi, carry):
            @plsc.parallel_loop(lo, hi, step=L, unroll=2, carry=carry)
            def accs(k, accs):
                accs = list(accs)
                idx_vec = idx_v[pl.ds(k, L)]
                for d in range(D):
                    dvec = jnp.full((L,), d, jnp.int32)
                    tcol = plsc.load_gather(tbl_v, [dvec, idx_vec])
                    fcol = feat_v[d, pl.ds(k, L)]
                    diff = fcol - tcol
                    accs[d % n_acc] = accs[d % n_acc] + diff * diff
                return tuple(accs)

            return accs

        ct.wait()
        ci.wait()
        cf0.wait()
        cf1.wait()
        accs = make_loop(0, b_per_w, (zero,) * n_acc)

        acc = zero
        for a in accs:
            acc = acc + a
        acc_v[...] = acc
        pltpu.sync_copy(acc_v, out_hbm.at[wid])

    partials = run(embedded_features.T, targets.astype(jnp.int32), hsa.T)
    return jnp.sum(partials) / B


# speedup vs baseline: 1.0064x; 1.0064x over previous
import functools

import jax
import jax.numpy as jnp
from jax import lax
from jax.experimental import pallas as pl
from jax.experimental.pallas import tpu as pltpu
from jax.experimental.pallas import tpu_sc as plsc


def kernel(embedded_features, targets, hsa):
    B, D = embedded_features.shape
    C = hsa.shape[0]
    info = plsc.get_sparse_core_info()
    NC, NS, L = info.num_cores, info.num_subcores, info.num_lanes
    NW = NC * NS
    b_per_w = B // NW

    mesh = plsc.VectorSubcoreMesh(core_axis_name="c", subcore_axis_name="s")

    @functools.partial(
        pl.kernel,
        mesh=mesh,
        compiler_params=pltpu.CompilerParams(needs_layout_passes=False),
        out_type=jax.ShapeDtypeStruct((NW, L), jnp.float32),
        scratch_types=[
            pltpu.VMEM((b_per_w,), jnp.int32),
            pltpu.VMEM((D, b_per_w), jnp.float32),
            pltpu.VMEM((D, C), jnp.float32),
            pltpu.VMEM((L,), jnp.float32),
            pltpu.SemaphoreType.DMA,
            pltpu.SemaphoreType.DMA,
            pltpu.SemaphoreType.DMA,
            pltpu.SemaphoreType.DMA,
        ],
    )
    def run(
        feat_hbm, tgt_hbm, hsa_hbm, out_hbm,
        idx_v, feat_v, tbl_v, acc_v, sem_t, sem_i, sem_f0, sem_f1,
    ):
        wid = lax.axis_index("s") * NC + lax.axis_index("c")
        base = wid * b_per_w
        half = b_per_w // 2
        ct = pltpu.async_copy(hsa_hbm, tbl_v, sem_t)
        ci = pltpu.async_copy(tgt_hbm.at[pl.ds(base, b_per_w)], idx_v, sem_i)
        cf0 = pltpu.async_copy(
            feat_hbm.at[:, pl.ds(base, half)], feat_v.at[:, pl.ds(0, half)], sem_f0
        )
        cf1 = pltpu.async_copy(
            feat_hbm.at[:, pl.ds(base + half, half)],
            feat_v.at[:, pl.ds(half, half)],
            sem_f1,
        )

        zero = jnp.zeros((L,), jnp.float32)
        n_acc = 8

        def make_loop(lo, hi, carry):
            @plsc.parallel_loop(lo, hi, step=L, carry=carry)
            def accs(k, accs):
                accs = list(accs)
                idx_vec = idx_v[pl.ds(k, L)]
                for d in range(D):
                    dvec = jnp.full((L,), d, jnp.int32)
                    tcol = plsc.load_gather(tbl_v, [dvec, idx_vec])
                    fcol = feat_v[d, pl.ds(k, L)]
                    diff = fcol - tcol
                    accs[d % n_acc] = accs[d % n_acc] + diff * diff
                return tuple(accs)

            return accs

        ct.wait()
        ci.wait()
        cf0.wait()
        cf1.wait()
        accs = make_loop(0, b_per_w, (zero,) * n_acc)

        acc = zero
        for a in accs:
            acc = acc + a
        acc_v[...] = acc
        pltpu.sync_copy(acc_v, out_hbm.at[wid])

    partials = run(embedded_features.T, targets.astype(jnp.int32), hsa.T)
    return jnp.sum(partials) / B
